# Initial kernel scaffold; baseline (speedup 1.0000x reference)
import jax
import jax.numpy as jnp
from jax.experimental import pallas as pl


def kernel(x, edge_index, edge_attr, in_W, in_b, Wk, bk, Wq, bq, Wv, bv, We, be, Wskip, conv_b, ln_g, ln_b, out_W):
    raise NotImplementedError("write your pallas kernel here")


# speedup vs baseline: 3.6351x; 3.6351x over previous
import functools

import jax
import jax.numpy as jnp
from jax import lax
from jax.experimental import pallas as pl
from jax.experimental.pallas import tpu as pltpu
from jax.experimental.pallas import tpu_sc as plsc


def _mm_bias_body(x_ref, w_ref, b_ref, o_ref):
    o_ref[...] = jnp.dot(x_ref[...], w_ref[...],
                         preferred_element_type=jnp.float32) + b_ref[...]


def _mm_bias(x, w, b, block_rows):
    m, kdim = x.shape
    dn = w.shape[1]
    return pl.pallas_call(
        _mm_bias_body,
        grid=(m // block_rows,),
        in_specs=[
            pl.BlockSpec((block_rows, kdim), lambda i: (i, 0)),
            pl.BlockSpec((kdim, dn), lambda i: (0, 0)),
            pl.BlockSpec((1, dn), lambda i: (0, 0)),
        ],
        out_specs=pl.BlockSpec((block_rows, dn), lambda i: (i, 0)),
        out_shape=jax.ShapeDtypeStruct((m, dn), jnp.float32),
    )(x, w, b.reshape(1, dn))


def _post_body(agg_ref, skip_ref, cb_ref, g_ref, b_ref, o_ref):
    out = agg_ref[0] + agg_ref[1] + skip_ref[...] + cb_ref[...]
    mu = jnp.mean(out, axis=-1, keepdims=True)
    var = jnp.mean((out - mu) ** 2, axis=-1, keepdims=True)
    out = (out - mu) / jnp.sqrt(var + 1e-5) * g_ref[...] + b_ref[...]
    o_ref[...] = out * 0.5 * (1.0 + lax.erf(out * 0.7071067811865476))


def _post(agg2, skip, cb, g, b, block_rows):
    n, d = skip.shape
    return pl.pallas_call(
        _post_body,
        grid=(n // block_rows,),
        in_specs=[
            pl.BlockSpec((2, block_rows, d), lambda i: (0, i, 0)),
            pl.BlockSpec((block_rows, d), lambda i: (i, 0)),
            pl.BlockSpec((1, d), lambda i: (0, 0)),
            pl.BlockSpec((1, d), lambda i: (0, 0)),
            pl.BlockSpec((1, d), lambda i: (0, 0)),
        ],
        out_specs=pl.BlockSpec((block_rows, d), lambda i: (i, 0)),
        out_shape=jax.ShapeDtypeStruct((n, d), jnp.float32),
    )(agg2, skip, cb.reshape(1, d), g.reshape(1, d), b.reshape(1, d))


_C = 80


def _edge_sc(k, q, v, e, src, dst):
    n, d = k.shape
    e_tot = src.shape[0]
    info = plsc.get_sparse_core_info()
    nc, ns = info.num_cores, info.num_subcores
    nw = nc * ns
    per_w = e_tot // nw
    n_chunks = per_w // _C
    rpt = -(-n // (8 * ns)) * 8
    n_pad = rpt * ns

    zrows = jnp.zeros((rpt, d), jnp.float32)
    mesh = plsc.VectorSubcoreMesh(core_axis_name="c", subcore_axis_name="s")

    @functools.partial(
        pl.kernel,
        mesh=mesh,
        out_type=jax.ShapeDtypeStruct((nc, n_pad, d), jnp.float32),
        scratch_types=[
            pltpu.VMEM((_C,), jnp.int32),
            pltpu.VMEM((_C,), jnp.int32),
            pltpu.VMEM((_C, d), jnp.float32),
            pltpu.VMEM((_C, d), jnp.float32),
            pltpu.VMEM((_C, d), jnp.float32),
            pltpu.VMEM((_C, d), jnp.float32),
            pltpu.VMEM_SHARED((n_pad, d), jnp.float32),
            pltpu.SemaphoreType.DMA,
            pltpu.SemaphoreType.DMA,
            pltpu.SemaphoreType.DMA,
        ],
    )
    def body(k_hbm, q_hbm, v_hbm, e_hbm, src_hbm, dst_hbm, z_hbm, out_hbm,
             src_v, dst_v, kb, qb, vb, eb, acc, s1, s2, s3):
        c = lax.axis_index("c")
        s = lax.axis_index("s")
        wid = c * ns + s
        pltpu.sync_copy(z_hbm, acc.at[pl.ds(s * rpt, rpt)])
        plsc.subcore_barrier()
        base = wid * per_w

        def chunk(g, carry):
            off = base + g * _C
            pltpu.sync_copy(src_hbm.at[pl.ds(off, _C)], src_v)
            pltpu.sync_copy(dst_hbm.at[pl.ds(off, _C)], dst_v)
            ck = pltpu.async_copy(k_hbm.at[dst_v], kb, s1)
            cq = pltpu.async_copy(q_hbm.at[src_v], qb, s2)
            cv = pltpu.async_copy(v_hbm.at[src_v], vb, s3)
            pltpu.sync_copy(e_hbm.at[pl.ds(off, _C)], eb)
            ck.wait()
            cq.wait()
            cv.wait()

            def row(r, rc):
                for j in range(d // 16):
                    sl = pl.ds(j * 16, 16)
                    t = kb[r, sl] + eb[r, sl] + qb[r, sl]
                    kb[r, sl] = vb[r, sl] / (1.0 + jnp.exp(-t))
                return rc

            lax.fori_loop(0, _C, row, 0)
            pltpu.sync_copy(kb, acc.at[dst_v], add=True)
            return carry

        lax.fori_loop(0, n_chunks, chunk, 0)
        plsc.subcore_barrier()
        pltpu.sync_copy(acc.at[pl.ds(s * rpt, rpt)],
                        out_hbm.at[c, pl.ds(s * rpt, rpt)])

    return body(k, q, v, e, src, dst, zrows)[:, :n, :]


def kernel(x, edge_index, edge_attr, in_W, in_b, Wk, bk, Wq, bq, Wv, bv,
           We, be, Wskip, conv_b, ln_g, ln_b, out_W):
    n, d = x.shape
    nlayers = Wk.shape[0]
    src = edge_index[0]
    dst = edge_index[1]
    zb = jnp.zeros((d,), jnp.float32)

    h = _mm_bias(x, in_W, in_b, 1000)
    for l in range(nlayers):
        kk = _mm_bias(h, Wk[l], bk[l], 1000)
        qq = _mm_bias(h, Wq[l], bq[l], 1000)
        vv = _mm_bias(h, Wv[l], bv[l], 1000)
        sk = _mm_bias(h, Wskip[l], zb, 1000)
        ee = _mm_bias(edge_attr, We[l], be[l], 2000)
        agg2 = _edge_sc(kk, qq, vv, ee, src, dst)
        h = _post(agg2, sk, conv_b[l], ln_g[l], ln_b[l], 1000)
    return _mm_bias(h, out_W, zb, 1000)


# speedup vs baseline: 4.4579x; 1.2264x over previous
import functools

import jax
import jax.numpy as jnp
from jax import lax
from jax.experimental import pallas as pl
from jax.experimental.pallas import tpu as pltpu
from jax.experimental.pallas import tpu_sc as plsc


def _mm_bias_body(x_ref, w_ref, b_ref, o_ref):
    o_ref[...] = jnp.dot(x_ref[...], w_ref[...],
                         preferred_element_type=jnp.float32) + b_ref[...]


def _mm_bias(x, w, b, block_rows):
    m, kdim = x.shape
    dn = w.shape[1]
    return pl.pallas_call(
        _mm_bias_body,
        grid=(m // block_rows,),
        in_specs=[
            pl.BlockSpec((block_rows, kdim), lambda i: (i, 0)),
            pl.BlockSpec((kdim, dn), lambda i: (0, 0)),
            pl.BlockSpec((1, dn), lambda i: (0, 0)),
        ],
        out_specs=pl.BlockSpec((block_rows, dn), lambda i: (i, 0)),
        out_shape=jax.ShapeDtypeStruct((m, dn), jnp.float32),
    )(x, w, b.reshape(1, dn))


def _post_body(agg_ref, skip_ref, cb_ref, g_ref, b_ref, o_ref):
    out = agg_ref[0] + agg_ref[1] + skip_ref[...] + cb_ref[...]
    mu = jnp.mean(out, axis=-1, keepdims=True)
    var = jnp.mean((out - mu) ** 2, axis=-1, keepdims=True)
    out = (out - mu) / jnp.sqrt(var + 1e-5) * g_ref[...] + b_ref[...]
    o_ref[...] = out * 0.5 * (1.0 + lax.erf(out * 0.7071067811865476))


def _post(agg2, skip, cb, g, b, block_rows):
    n, d = skip.shape
    return pl.pallas_call(
        _post_body,
        grid=(n // block_rows,),
        in_specs=[
            pl.BlockSpec((2, block_rows, d), lambda i: (0, i, 0)),
            pl.BlockSpec((block_rows, d), lambda i: (i, 0)),
            pl.BlockSpec((1, d), lambda i: (0, 0)),
            pl.BlockSpec((1, d), lambda i: (0, 0)),
            pl.BlockSpec((1, d), lambda i: (0, 0)),
        ],
        out_specs=pl.BlockSpec((block_rows, d), lambda i: (i, 0)),
        out_shape=jax.ShapeDtypeStruct((n, d), jnp.float32),
    )(agg2, skip, cb.reshape(1, d), g.reshape(1, d), b.reshape(1, d))


_C = 40


def _edge_sc(k, q, v, e, src, dst):
    n, d = k.shape
    e_tot = src.shape[0]
    info = plsc.get_sparse_core_info()
    nc, ns = info.num_cores, info.num_subcores
    nw = nc * ns
    per_w = e_tot // nw
    n_chunks = per_w // _C
    assert per_w % _C == 0 and n_chunks % 2 == 0
    rpt = -(-n // (8 * ns)) * 8
    n_pad = rpt * ns

    zrows = jnp.zeros((rpt, d), jnp.float32)
    mesh = plsc.VectorSubcoreMesh(core_axis_name="c", subcore_axis_name="s")

    @functools.partial(
        pl.kernel,
        mesh=mesh,
        out_type=jax.ShapeDtypeStruct((nc, n_pad, d), jnp.float32),
        scratch_types=[
            pltpu.VMEM((_C,), jnp.int32),
            pltpu.VMEM((_C,), jnp.int32),
            pltpu.VMEM((_C,), jnp.int32),
            pltpu.VMEM((_C,), jnp.int32),
            pltpu.VMEM((_C, d), jnp.float32),
            pltpu.VMEM((_C, d), jnp.float32),
            pltpu.VMEM((_C, d), jnp.float32),
            pltpu.VMEM((_C, d), jnp.float32),
            pltpu.VMEM((_C, d), jnp.float32),
            pltpu.VMEM((_C, d), jnp.float32),
            pltpu.VMEM((_C, d), jnp.float32),
            pltpu.VMEM((_C, d), jnp.float32),
            pltpu.VMEM_SHARED((n_pad, d), jnp.float32),
            pltpu.SemaphoreType.DMA,
            pltpu.SemaphoreType.DMA,
        ],
    )
    def body(k_hbm, q_hbm, v_hbm, e_hbm, src_hbm, dst_hbm, z_hbm, out_hbm,
             src0, dst0, src1, dst1, kb0, qb0, vb0, eb0, kb1, qb1, vb1, eb1,
             acc, s0, s1):
        c = lax.axis_index("c")
        s = lax.axis_index("s")
        wid = c * ns + s
        pltpu.sync_copy(z_hbm, acc.at[pl.ds(s * rpt, rpt)])
        plsc.subcore_barrier()
        base = wid * per_w

        bufs = ((src0, dst0, kb0, qb0, vb0, eb0, s0),
                (src1, dst1, kb1, qb1, vb1, eb1, s1))

        def issue(off, b):
            srcv, dstv, kb, qb, vb, eb, sem = bufs[b]
            pltpu.sync_copy(src_hbm.at[pl.ds(off, _C)], srcv)
            pltpu.sync_copy(dst_hbm.at[pl.ds(off, _C)], dstv)
            pltpu.async_copy(k_hbm.at[dstv], kb, sem)
            pltpu.async_copy(q_hbm.at[srcv], qb, sem)
            pltpu.async_copy(v_hbm.at[srcv], vb, sem)
            pltpu.async_copy(e_hbm.at[pl.ds(off, _C)], eb, sem)

        def finish(b):
            srcv, dstv, kb, qb, vb, eb, sem = bufs[b]
            pltpu.make_async_copy(k_hbm.at[dstv], kb, sem).wait()
            pltpu.make_async_copy(q_hbm.at[srcv], qb, sem).wait()
            pltpu.make_async_copy(v_hbm.at[srcv], vb, sem).wait()
            pltpu.make_async_copy(e_hbm.at[pl.ds(0, _C)], eb, sem).wait()

            def row(r, rc):
                for j in range(d // 16):
                    sl = pl.ds(j * 16, 16)
                    t = kb[r, sl] + eb[r, sl] + qb[r, sl]
                    kb[r, sl] = vb[r, sl] / (1.0 + jnp.exp(-t))
                return rc

            lax.fori_loop(0, _C, row, 0)
            pltpu.sync_copy(kb, acc.at[dstv], add=True)

        issue(base, 0)

        def two(i, carry):
            g = i * 2
            issue(base + (g + 1) * _C, 1)
            finish(0)
            issue(base + (g + 2) * _C, 0)
            finish(1)
            return carry

        lax.fori_loop(0, n_chunks // 2 - 1, two, 0)
        issue(base + (n_chunks - 1) * _C, 1)
        finish(0)
        finish(1)
        plsc.subcore_barrier()
        pltpu.sync_copy(acc.at[pl.ds(s * rpt, rpt)],
                        out_hbm.at[c, pl.ds(s * rpt, rpt)])

    return body(k, q, v, e, src, dst, zrows)[:, :n, :]


def kernel(x, edge_index, edge_attr, in_W, in_b, Wk, bk, Wq, bq, Wv, bv,
           We, be, Wskip, conv_b, ln_g, ln_b, out_W):
    n, d = x.shape
    nlayers = Wk.shape[0]
    src = edge_index[0]
    dst = edge_index[1]
    zb = jnp.zeros((d,), jnp.float32)

    h = _mm_bias(x, in_W, in_b, 1000)
    for l in range(nlayers):
        kk = _mm_bias(h, Wk[l], bk[l], 1000)
        qq = _mm_bias(h, Wq[l], bq[l], 1000)
        vv = _mm_bias(h, Wv[l], bv[l], 1000)
        sk = _mm_bias(h, Wskip[l], zb, 1000)
        ee = _mm_bias(edge_attr, We[l], be[l], 2000)
        agg2 = _edge_sc(kk, qq, vv, ee, src, dst)
        h = _post(agg2, sk, conv_b[l], ln_g[l], ln_b[l], 1000)
    return _mm_bias(h, out_W, zb, 1000)
